# Initial kernel scaffold; baseline (speedup 1.0000x reference)
import jax
import jax.numpy as jnp
from jax.experimental import pallas as pl


def kernel(hidden_states, router_weight, gate_up_proj, down_proj):
    raise NotImplementedError("write your pallas kernel here")


# speedup vs baseline: 1.0828x; 1.0828x over previous
import functools

import jax
import jax.numpy as jnp
from jax.experimental import pallas as pl
from jax.experimental.pallas import tpu as pltpu

E = 8
TOPK = 2
H = 1024
FF = 4096
T = 16
FB = 512


def _moe_kernel(x_ref, rw_ref, wg_ref, wu_ref, wd_ref,
                out_ref, logits_ref, scores_ref):
    e = pl.program_id(0)
    f = pl.program_id(1)

    @pl.when(jnp.logical_and(e == 0, f == 0))
    def _prologue():
        x = x_ref[...]
        logits = jnp.dot(x, rw_ref[...].T, preferred_element_type=jnp.float32)
        logits_ref[...] = logits
        idx = jax.lax.broadcasted_iota(jnp.int32, (T, E), 1)
        m1 = jnp.max(logits, axis=1, keepdims=True)
        i1 = jnp.min(jnp.where(logits == m1, idx, E), axis=1, keepdims=True)
        mask1 = idx == i1
        l2 = jnp.where(mask1, -jnp.inf, logits)
        m2 = jnp.max(l2, axis=1, keepdims=True)
        i2 = jnp.min(jnp.where(l2 == m2, idx, E), axis=1, keepdims=True)
        mask2 = idx == i2
        scores_ref[...] = jnp.where(mask1 | mask2,
                                    jax.nn.sigmoid(logits), 0.0)
        out_ref[...] = jnp.zeros_like(out_ref)

    col = jax.lax.broadcasted_iota(jnp.int32, (T, E), 1)
    s = jnp.sum(jnp.where(col == e, scores_ref[...], 0.0), axis=1,
                keepdims=True)
    xs = x_ref[...] * s
    gate = jnp.dot(xs, wg_ref[0], preferred_element_type=jnp.float32)
    up = jnp.dot(xs, wu_ref[0], preferred_element_type=jnp.float32)
    act = up * (gate * jax.nn.sigmoid(gate))
    out_ref[...] += jnp.dot(act, wd_ref[0], preferred_element_type=jnp.float32)


def _moe(hidden_states, router_weight, gate_up_proj, down_proj, interpret=False):
    nf = FF // FB
    out, logits = pl.pallas_call(
        _moe_kernel,
        grid=(E, nf),
        in_specs=[
            pl.BlockSpec((T, H), lambda e, f: (0, 0)),
            pl.BlockSpec((E, H), lambda e, f: (0, 0)),
            pl.BlockSpec((1, H, FB), lambda e, f: (e, 0, f)),
            pl.BlockSpec((1, H, FB), lambda e, f: (e, 0, f + FF // FB)),
            pl.BlockSpec((1, FB, H), lambda e, f: (e, f, 0)),
        ],
        out_specs=[
            pl.BlockSpec((T, H), lambda e, f: (0, 0)),
            pl.BlockSpec((T, E), lambda e, f: (0, 0)),
        ],
        out_shape=[
            jax.ShapeDtypeStruct((T, H), jnp.float32),
            jax.ShapeDtypeStruct((T, E), jnp.float32),
        ],
        scratch_shapes=[pltpu.VMEM((T, E), jnp.float32)],
        interpret=interpret,
    )(hidden_states, router_weight, gate_up_proj, gate_up_proj, down_proj)
    return out, logits


def kernel(hidden_states, router_weight, gate_up_proj, down_proj):
    return _moe(hidden_states.reshape(-1, H), router_weight,
                gate_up_proj, down_proj)
